# Initial kernel scaffold; baseline (speedup 1.0000x reference)
import jax
import jax.numpy as jnp
from jax.experimental import pallas as pl


def kernel(x, edge_index, W1, b1, Wih, Whh, bih, bhh, W2, b2):
    raise NotImplementedError("write your pallas kernel here")


# speedup vs baseline: 5.0830x; 5.0830x over previous
import functools

import jax
import jax.numpy as jnp
from jax import lax
from jax.experimental import pallas as pl
from jax.experimental.pallas import tpu as pltpu
from jax.experimental.pallas import tpu_sc as plsc

N = 10000
NP = 10240
D = 128
NC = 2
NS = 16
NW = NC * NS
C = 128
ROWS_PER_TILE = NP // NS

_mesh = plsc.VectorSubcoreMesh(core_axis_name="c", subcore_axis_name="s")


def _make_spmm(chunks_per_worker):
    epw = C * chunks_per_worker

    @functools.partial(
        pl.kernel,
        out_type=jax.ShapeDtypeStruct((NC, NP, D), jnp.float32),
        mesh=_mesh,
        scratch_types=[
            pltpu.VMEM((C,), jnp.int32),
            pltpu.VMEM((C,), jnp.int32),
            pltpu.VMEM((C, D), jnp.float32),
            pltpu.VMEM_SHARED((NP, D), jnp.float32),
            pltpu.SemaphoreType.DMA,
        ],
    )
    def spmm(hp_hbm, src_hbm, dst_hbm, zeros_hbm, out_hbm,
             src_v, dst_v, rows_v, acc, sem):
        c = lax.axis_index("c")
        s = lax.axis_index("s")
        wid = s * NC + c
        r0 = s * ROWS_PER_TILE
        pltpu.sync_copy(zeros_hbm.at[pl.ds(r0, ROWS_PER_TILE)],
                        acc.at[pl.ds(r0, ROWS_PER_TILE)])
        plsc.subcore_barrier()

        base = wid * epw

        def body(k, carry):
            off = base + k * C
            pltpu.sync_copy(src_hbm.at[pl.ds(off, C)], src_v)
            pltpu.sync_copy(dst_hbm.at[pl.ds(off, C)], dst_v)
            pltpu.async_copy(hp_hbm.at[src_v], rows_v, sem).wait()
            pltpu.sync_copy(rows_v, acc.at[dst_v], add=True)
            return carry

        lax.fori_loop(0, chunks_per_worker, body, 0)
        plsc.subcore_barrier()
        pltpu.sync_copy(acc.at[pl.ds(r0, ROWS_PER_TILE)],
                        out_hbm.at[c, pl.ds(r0, ROWS_PER_TILE)])

    return spmm


BR = 2000
_GRID = N // BR


def _l2n(h):
    return h / (jnp.sqrt(jnp.sum(h * h, axis=1, keepdims=True)) + 1e-8)


def _enc_body(x_ref, w1_ref, b1_ref, degp_ref, h_ref, hp_ref, a_ref):
    h = jnp.maximum(
        jnp.dot(x_ref[...], w1_ref[...], preferred_element_type=jnp.float32)
        + b1_ref[...], 0.0)
    h = _l2n(h)
    a = lax.rsqrt(degp_ref[0] + degp_ref[1] + 1.0)
    h_ref[...] = h
    a_ref[...] = a
    hp_ref[...] = h * a


def _gru_body(p_ref, h_ref, a_ref, wih_ref, whh_ref, bih_ref, bhh_ref, z_ref,
              hn_ref, hpn_ref, zn_ref):
    a = a_ref[...]
    h = h_ref[...]
    agg = (p_ref[0] + p_ref[1]) * a
    gi = jnp.dot(agg, wih_ref[...], preferred_element_type=jnp.float32) + bih_ref[...]
    gh = jnp.dot(h, whh_ref[...], preferred_element_type=jnp.float32) + bhh_ref[...]
    r = jax.nn.sigmoid(gi[:, :D] + gh[:, :D])
    zg = jax.nn.sigmoid(gi[:, D:2 * D] + gh[:, D:2 * D])
    n = jnp.tanh(gi[:, 2 * D:] + r * gh[:, 2 * D:])
    hn = _l2n((1.0 - zg) * n + zg * h)
    hn_ref[...] = hn
    hpn_ref[...] = hn * a
    zn_ref[...] = jnp.maximum(z_ref[...], hn)


def _head_body(z_ref, w2_ref, b2_ref, out_ref):
    out_ref[...] = (
        jnp.dot(z_ref[...], w2_ref[...], preferred_element_type=jnp.float32)
        + b2_ref[...])


def _row_spec(width):
    return pl.BlockSpec((BR, width), lambda i: (i, 0))


def _full_spec(shape):
    return pl.BlockSpec(shape, lambda i: tuple(0 for _ in shape))


_encoder = pl.pallas_call(
    _enc_body,
    grid=(_GRID,),
    in_specs=[
        _row_spec(D),
        _full_spec((D, D)),
        _full_spec((1, D)),
        pl.BlockSpec((NC, BR, D), lambda i: (0, i, 0)),
    ],
    out_specs=[_row_spec(D), _row_spec(D), _row_spec(D)],
    out_shape=[jax.ShapeDtypeStruct((N, D), jnp.float32)] * 3,
)

_gru = pl.pallas_call(
    _gru_body,
    grid=(_GRID,),
    in_specs=[
        pl.BlockSpec((NC, BR, D), lambda i: (0, i, 0)),
        _row_spec(D),
        _row_spec(D),
        _full_spec((D, 3 * D)),
        _full_spec((D, 3 * D)),
        _full_spec((1, 3 * D)),
        _full_spec((1, 3 * D)),
        _row_spec(D),
    ],
    out_specs=[_row_spec(D), _row_spec(D), _row_spec(D)],
    out_shape=[jax.ShapeDtypeStruct((N, D), jnp.float32)] * 3,
)

_head = pl.pallas_call(
    _head_body,
    grid=(_GRID,),
    in_specs=[_row_spec(D), _full_spec((D, D)), _full_spec((1, D))],
    out_specs=_row_spec(D),
    out_shape=jax.ShapeDtypeStruct((N, D), jnp.float32),
)


def kernel(x, edge_index, W1, b1, Wih, Whh, bih, bhh, W2, b2):
    E = edge_index.shape[1]
    chunks_per_worker = -(-E // (C * NW))
    e_pad = C * NW * chunks_per_worker
    spmm = _make_spmm(chunks_per_worker)

    src = edge_index[0].astype(jnp.int32)
    dst = edge_index[1].astype(jnp.int32)
    pad = e_pad - E
    src_p = jnp.concatenate([src, jnp.zeros((pad,), jnp.int32)])
    dst_p = jnp.concatenate([dst, jnp.full((pad,), N, jnp.int32)])

    zeros_nd = jnp.zeros((NP, D), jnp.float32)
    ones_nd = jnp.ones((N, D), jnp.float32)

    degp = spmm(ones_nd, src_p, dst_p, zeros_nd)
    h, hp, a = _encoder(x, W1, b1.reshape(1, D), degp)
    z = h
    bih2 = bih.reshape(1, 3 * D)
    bhh2 = bhh.reshape(1, 3 * D)
    for _ in range(4):
        p = spmm(hp, src_p, dst_p, zeros_nd)
        h, hp, z = _gru(p, h, a, Wih, Whh, bih2, bhh2, z)
    return _head(z, W2, b2.reshape(1, D))
